# Initial kernel scaffold; baseline (speedup 1.0000x reference)
import jax
import jax.numpy as jnp
from jax.experimental import pallas as pl


def kernel(x, edge_index, W, b):
    raise NotImplementedError("write your pallas kernel here")


# speedup vs baseline: 13.5636x; 13.5636x over previous
import functools

import jax
import jax.numpy as jnp
from jax import lax
from jax.experimental import pallas as pl
from jax.experimental.pallas import tpu as pltpu
from jax.experimental.pallas import tpu_sc as plsc

N_NODES = 10000
IN_CH = 128
OUT_CH = 128

NC = 2
NS = 16
NW = NC * NS
CHUNK = 128
NPAD = 10240
ROWS_PER_TILE = NPAD // NS

N_EDGES = 320000
N_CHUNKS_W = -(-N_EDGES // (NW * CHUNK))
E_PER_W = N_CHUNKS_W * CHUNK
EPAD = E_PER_W * NW
E_PER_C = EPAD // NC

BLK = 1024
GRID = NPAD // BLK

_mesh = lambda: plsc.VectorSubcoreMesh(
    core_axis_name="c", subcore_axis_name="s", num_cores=NC, num_subcores=NS)


@functools.partial(
    pl.kernel,
    out_type=jax.ShapeDtypeStruct((NC, NPAD), jnp.float32),
    mesh=_mesh(),
    scratch_types=[
        pltpu.VMEM((CHUNK,), jnp.int32),
        pltpu.VMEM((CHUNK,), jnp.float32),
        pltpu.VMEM_SHARED((NPAD,), jnp.float32),
    ],
)
def _deg_kernel(row_hbm, hist_hbm, idxv, onesv, acc):
    c = lax.axis_index("c")
    s = lax.axis_index("s")

    for k in range(CHUNK // 16):
        onesv[pl.ds(k * 16, 16)] = jnp.zeros((16,), jnp.float32)
    base_r = s * ROWS_PER_TILE
    @pl.loop(0, ROWS_PER_TILE // CHUNK)
    def _zero(i):
        pltpu.sync_copy(onesv, acc.at[pl.ds(base_r + i * CHUNK, CHUNK)])
    for k in range(CHUNK // 16):
        onesv[pl.ds(k * 16, 16)] = jnp.ones((16,), jnp.float32)
    plsc.subcore_barrier()

    base_e = c * E_PER_C + s * E_PER_W
    @pl.loop(0, N_CHUNKS_W)
    def _hist(j):
        pltpu.sync_copy(row_hbm.at[pl.ds(base_e + j * CHUNK, CHUNK)], idxv)
        pltpu.sync_copy(onesv, acc.at[idxv], add=True)
    plsc.subcore_barrier()

    @pl.loop(0, ROWS_PER_TILE // CHUNK)
    def _wb(i):
        off = base_r + i * CHUNK
        pltpu.sync_copy(acc.at[pl.ds(off, CHUNK)], onesv)
        pltpu.sync_copy(onesv, hist_hbm.at[c, pl.ds(off, CHUNK)])


@functools.partial(
    pl.kernel,
    out_type=jax.ShapeDtypeStruct((NC, NPAD, OUT_CH), jnp.float32),
    mesh=_mesh(),
    scratch_types=[
        pltpu.VMEM((CHUNK,), jnp.int32),
        pltpu.VMEM((CHUNK,), jnp.int32),
        pltpu.VMEM((CHUNK, OUT_CH), jnp.float32),
        pltpu.VMEM((16, OUT_CH), jnp.float32),
        pltpu.VMEM_SHARED((NPAD, OUT_CH), jnp.float32),
        pltpu.SemaphoreType.DMA,
    ],
)
def _edge_kernel(sup_hbm, col_hbm, row_hbm, out_hbm,
                 colv, rowv, rows, ztile, acc, sem):
    c = lax.axis_index("c")
    s = lax.axis_index("s")

    for r in range(16):
        for k in range(OUT_CH // 16):
            ztile[r, pl.ds(k * 16, 16)] = jnp.zeros((16,), jnp.float32)
    base_r = s * ROWS_PER_TILE
    @pl.loop(0, ROWS_PER_TILE // 16)
    def _zero(i):
        pltpu.sync_copy(ztile, acc.at[pl.ds(base_r + i * 16, 16)])
    plsc.subcore_barrier()

    base_e = c * E_PER_C + s * E_PER_W
    @pl.loop(0, N_CHUNKS_W)
    def _edges(j):
        off = base_e + j * CHUNK
        pltpu.sync_copy(col_hbm.at[pl.ds(off, CHUNK)], colv)
        pltpu.sync_copy(row_hbm.at[pl.ds(off, CHUNK)], rowv)
        pltpu.async_copy(sup_hbm.at[colv], rows, sem).wait()
        pltpu.sync_copy(rows, acc.at[rowv], add=True)
    plsc.subcore_barrier()

    @pl.loop(0, ROWS_PER_TILE // CHUNK)
    def _wb(i):
        off = base_r + i * CHUNK
        pltpu.sync_copy(acc.at[pl.ds(off, CHUNK)], rows)
        pltpu.sync_copy(rows, out_hbm.at[c, pl.ds(off, CHUNK)])


def _support_body(x_ref, wt_ref, b_ref, h0_ref, h1_ref, sup_ref, dis_ref):
    deg = 1.0 + h0_ref[...] + h1_ref[...]
    dis = lax.rsqrt(deg)
    s = jnp.dot(x_ref[...], wt_ref[...],
                preferred_element_type=jnp.float32) + b_ref[...]
    sup_ref[...] = dis * s
    dis_ref[...] = dis


def _support_call(x_pad, wt, b2, h0, h1):
    return pl.pallas_call(
        _support_body,
        grid=(GRID,),
        in_specs=[
            pl.BlockSpec((BLK, IN_CH), lambda i: (i, 0)),
            pl.BlockSpec((IN_CH, OUT_CH), lambda i: (0, 0)),
            pl.BlockSpec((1, OUT_CH), lambda i: (0, 0)),
            pl.BlockSpec((BLK, 1), lambda i: (i, 0)),
            pl.BlockSpec((BLK, 1), lambda i: (i, 0)),
        ],
        out_specs=[
            pl.BlockSpec((BLK, OUT_CH), lambda i: (i, 0)),
            pl.BlockSpec((BLK, 1), lambda i: (i, 0)),
        ],
        out_shape=[
            jax.ShapeDtypeStruct((NPAD, OUT_CH), jnp.float32),
            jax.ShapeDtypeStruct((NPAD, 1), jnp.float32),
        ],
    )(x_pad, wt, b2, h0, h1)


def _combine_body(p0_ref, p1_ref, sup_ref, dis_ref, out_ref):
    out_ref[...] = dis_ref[...] * (p0_ref[...] + p1_ref[...] + sup_ref[...])


def _combine_call(p0, p1, sup, dis):
    return pl.pallas_call(
        _combine_body,
        grid=(GRID,),
        in_specs=[
            pl.BlockSpec((BLK, OUT_CH), lambda i: (i, 0)),
            pl.BlockSpec((BLK, OUT_CH), lambda i: (i, 0)),
            pl.BlockSpec((BLK, OUT_CH), lambda i: (i, 0)),
            pl.BlockSpec((BLK, 1), lambda i: (i, 0)),
        ],
        out_specs=pl.BlockSpec((BLK, OUT_CH), lambda i: (i, 0)),
        out_shape=jax.ShapeDtypeStruct((NPAD, OUT_CH), jnp.float32),
    )(p0, p1, sup, dis)


def kernel(x, edge_index, W, b):
    ei = edge_index.astype(jnp.int32)
    row = jnp.pad(ei[0], (0, EPAD - N_EDGES), constant_values=N_NODES)
    col = jnp.pad(ei[1], (0, EPAD - N_EDGES), constant_values=0)

    hist = _deg_kernel(row)
    h0 = hist[0].reshape(NPAD, 1)
    h1 = hist[1].reshape(NPAD, 1)

    x_pad = jnp.pad(x, ((0, NPAD - N_NODES), (0, 0)))
    wt = W.T
    b2 = b.reshape(1, OUT_CH)
    sup, dis = _support_call(x_pad, wt, b2, h0, h1)

    partials = _edge_kernel(sup, col, row)
    out = _combine_call(partials[0], partials[1], sup, dis)
    return out[:N_NODES]


# speedup vs baseline: 15.3642x; 1.1328x over previous
import functools

import jax
import jax.numpy as jnp
from jax import lax
from jax.experimental import pallas as pl
from jax.experimental.pallas import tpu as pltpu
from jax.experimental.pallas import tpu_sc as plsc

N_NODES = 10000
IN_CH = 128
OUT_CH = 128

NC = 2
NS = 16
NW = NC * NS
CHUNK = 128
NPAD = 10240
ROWS_PER_TILE = NPAD // NS

N_EDGES = 320000
N_CHUNKS_W = 80
E_PER_W = N_CHUNKS_W * CHUNK
EPAD = E_PER_W * NW
E_PER_C = EPAD // NC

BLK = 1024
GRID = NPAD // BLK

_mesh = lambda: plsc.VectorSubcoreMesh(
    core_axis_name="c", subcore_axis_name="s", num_cores=NC, num_subcores=NS)


@functools.partial(
    pl.kernel,
    out_type=jax.ShapeDtypeStruct((NC, NPAD), jnp.float32),
    mesh=_mesh(),
    scratch_types=[
        pltpu.VMEM((CHUNK,), jnp.int32),
        pltpu.VMEM((CHUNK,), jnp.float32),
        pltpu.VMEM_SHARED((NPAD,), jnp.float32),
    ],
)
def _deg_kernel(row_hbm, hist_hbm, idxv, onesv, acc):
    c = lax.axis_index("c")
    s = lax.axis_index("s")

    for k in range(CHUNK // 16):
        onesv[pl.ds(k * 16, 16)] = jnp.zeros((16,), jnp.float32)
    base_r = s * ROWS_PER_TILE
    @pl.loop(0, ROWS_PER_TILE // CHUNK)
    def _zero(i):
        pltpu.sync_copy(onesv, acc.at[pl.ds(base_r + i * CHUNK, CHUNK)])
    for k in range(CHUNK // 16):
        onesv[pl.ds(k * 16, 16)] = jnp.ones((16,), jnp.float32)
    plsc.subcore_barrier()

    base_e = c * E_PER_C + s * E_PER_W
    @pl.loop(0, N_CHUNKS_W)
    def _hist(j):
        pltpu.sync_copy(row_hbm.at[pl.ds(base_e + j * CHUNK, CHUNK)], idxv)
        pltpu.sync_copy(onesv, acc.at[idxv], add=True)
    plsc.subcore_barrier()

    @pl.loop(0, ROWS_PER_TILE // CHUNK)
    def _wb(i):
        off = base_r + i * CHUNK
        pltpu.sync_copy(acc.at[pl.ds(off, CHUNK)], onesv)
        pltpu.sync_copy(onesv, hist_hbm.at[c, pl.ds(off, CHUNK)])


@functools.partial(
    pl.kernel,
    out_type=jax.ShapeDtypeStruct((NC, NPAD, OUT_CH), jnp.float32),
    mesh=_mesh(),
    scratch_types=[
        pltpu.VMEM((CHUNK,), jnp.int32),
        pltpu.VMEM((CHUNK,), jnp.int32),
        pltpu.VMEM((N_CHUNKS_W, CHUNK), jnp.int32),
        pltpu.VMEM((CHUNK, OUT_CH), jnp.float32),
        pltpu.VMEM((CHUNK, OUT_CH), jnp.float32),
        pltpu.VMEM((8, OUT_CH), jnp.float32),
        pltpu.VMEM_SHARED((NPAD, OUT_CH), jnp.float32),
        pltpu.SemaphoreType.DMA,
        pltpu.SemaphoreType.DMA,
    ],
)
def _edge_kernel(sup_hbm, col_hbm, row_hbm, out_hbm,
                 colv0, colv1, rowv, buf0, buf1, ztile, acc, sem0, sem1):
    c = lax.axis_index("c")
    s = lax.axis_index("s")
    wid = c * NS + s

    for r in range(8):
        for k in range(OUT_CH // 16):
            ztile[r, pl.ds(k * 16, 16)] = jnp.zeros((16,), jnp.float32)
    base_r = s * ROWS_PER_TILE
    @pl.loop(0, ROWS_PER_TILE // 8)
    def _zero(i):
        pltpu.sync_copy(ztile, acc.at[pl.ds(base_r + i * 8, 8)])

    pltpu.sync_copy(row_hbm.at[wid], rowv)
    plsc.subcore_barrier()

    colvs = (colv0, colv1)
    bufs = (buf0, buf1)
    sems = (sem0, sem1)
    pltpu.sync_copy(col_hbm.at[wid, 0], colv0)
    pltpu.async_copy(sup_hbm.at[colv0], buf0, sem0)
    @pl.loop(0, N_CHUNKS_W // 2)
    def _pair(i):
        j0 = 2 * i
        for p in range(2):
            j = j0 + p
            jn = lax.rem(j + 1, N_CHUNKS_W)
            pltpu.sync_copy(col_hbm.at[wid, jn], colvs[1 - p])
            pltpu.async_copy(sup_hbm.at[colvs[1 - p]], bufs[1 - p], sems[1 - p])
            pltpu.make_async_copy(sup_hbm.at[colvs[p]], bufs[p], sems[p]).wait()
            pltpu.sync_copy(bufs[p], acc.at[rowv.at[j]], add=True)
    pltpu.make_async_copy(sup_hbm.at[colv0], buf0, sem0).wait()
    plsc.subcore_barrier()

    @pl.loop(0, ROWS_PER_TILE // CHUNK)
    def _wb(i):
        off = base_r + i * CHUNK
        pltpu.sync_copy(acc.at[pl.ds(off, CHUNK)], buf0)
        pltpu.sync_copy(buf0, out_hbm.at[c, pl.ds(off, CHUNK)])


def _support_body(x_ref, wt_ref, b_ref, h0_ref, h1_ref, sup_ref, dis_ref):
    deg = 1.0 + h0_ref[...] + h1_ref[...]
    dis = lax.rsqrt(deg)
    s = jnp.dot(x_ref[...], wt_ref[...],
                preferred_element_type=jnp.float32) + b_ref[...]
    sup_ref[...] = dis * s
    dis_ref[...] = dis


def _support_call(x_pad, wt, b2, h0, h1):
    return pl.pallas_call(
        _support_body,
        grid=(GRID,),
        in_specs=[
            pl.BlockSpec((BLK, IN_CH), lambda i: (i, 0)),
            pl.BlockSpec((IN_CH, OUT_CH), lambda i: (0, 0)),
            pl.BlockSpec((1, OUT_CH), lambda i: (0, 0)),
            pl.BlockSpec((BLK, 1), lambda i: (i, 0)),
            pl.BlockSpec((BLK, 1), lambda i: (i, 0)),
        ],
        out_specs=[
            pl.BlockSpec((BLK, OUT_CH), lambda i: (i, 0)),
            pl.BlockSpec((BLK, 1), lambda i: (i, 0)),
        ],
        out_shape=[
            jax.ShapeDtypeStruct((NPAD, OUT_CH), jnp.float32),
            jax.ShapeDtypeStruct((NPAD, 1), jnp.float32),
        ],
    )(x_pad, wt, b2, h0, h1)


def _combine_body(p0_ref, p1_ref, sup_ref, dis_ref, out_ref):
    out_ref[...] = dis_ref[...] * (p0_ref[...] + p1_ref[...] + sup_ref[...])


def _combine_call(p0, p1, sup, dis):
    return pl.pallas_call(
        _combine_body,
        grid=(GRID,),
        in_specs=[
            pl.BlockSpec((BLK, OUT_CH), lambda i: (i, 0)),
            pl.BlockSpec((BLK, OUT_CH), lambda i: (i, 0)),
            pl.BlockSpec((BLK, OUT_CH), lambda i: (i, 0)),
            pl.BlockSpec((BLK, 1), lambda i: (i, 0)),
        ],
        out_specs=pl.BlockSpec((BLK, OUT_CH), lambda i: (i, 0)),
        out_shape=jax.ShapeDtypeStruct((NPAD, OUT_CH), jnp.float32),
    )(p0, p1, sup, dis)


def kernel(x, edge_index, W, b):
    ei = edge_index.astype(jnp.int32)
    row = jnp.pad(ei[0], (0, EPAD - N_EDGES), constant_values=N_NODES)
    col = jnp.pad(ei[1], (0, EPAD - N_EDGES), constant_values=0)

    hist = _deg_kernel(row)
    h0 = hist[0].reshape(NPAD, 1)
    h1 = hist[1].reshape(NPAD, 1)

    x_pad = jnp.pad(x, ((0, NPAD - N_NODES), (0, 0)))
    wt = W.T
    b2 = b.reshape(1, OUT_CH)
    sup, dis = _support_call(x_pad, wt, b2, h0, h1)

    col3 = col.reshape(NW, N_CHUNKS_W, CHUNK)
    row3 = row.reshape(NW, N_CHUNKS_W, CHUNK)
    partials = _edge_kernel(sup, col3, row3)
    out = _combine_call(partials[0], partials[1], sup, dis)
    return out[:N_NODES]
